# baseline (speedup 1.0000x reference)
import jax
import jax.numpy as jnp
from jax import lax
from jax.experimental import pallas as pl
from jax.experimental.pallas import tpu as pltpu
from jax.experimental.pallas import tpu_sc as plsc

_B, _D, _C = 1024, 32, 100000
_TSIM = 0.8
_EPS = 1e-8
_TILE = 1024
_NT = 98
_CPAD = _NT * _TILE
_NEG = -3.0e38
_NW = 32
_R = _B // _NW


def _phase_a(mv_ref, pat_ref, usage_ref, sc_ref,
             idx_ref, sim_ref, win_ref, wrow_ref, wcol_ref, nuse_ref,
             pay_ref, load_ref, outer_ref,
             bval, bidx, bsim, blru, bsc, nnz):
  pid = pl.program_id(0)

  @pl.when(pid == 0)
  def _init():
    bval[...] = jnp.full((_B, 1), _NEG, jnp.float32)
    bidx[...] = jnp.zeros((_B, 1), jnp.float32)
    bsim[...] = jnp.zeros((_B, 1), jnp.float32)
    blru[...] = jnp.zeros((_B, 1), jnp.float32)
    bsc[...] = jnp.zeros((_B, 1), jnp.float32)
    nnz[...] = jnp.zeros((1, 1), jnp.float32)

  mv = mv_ref[...]
  vn = mv / jnp.maximum(
      jnp.sqrt(jnp.sum(mv * mv, axis=1, keepdims=True)), _EPS)
  p = pat_ref[...]
  pn = p / jnp.maximum(
      jnp.sqrt(jnp.sum(p * p, axis=1, keepdims=True)), _EPS)
  sim = lax.dot_general(vn, pn, (((1,), (1,)), ((), ())),
                        preferred_element_type=jnp.float32)

  usage = usage_ref[0]
  lru01 = (1.0 / (1.0 + usage)) * 0.1
  scv = sc_ref[0]

  coli = lax.broadcasted_iota(jnp.int32, (_B, _TILE), 1)
  valid = pid * _TILE + coli < _C
  lru_b = jnp.broadcast_to(lru01, (_B, _TILE))
  sel = jnp.where(sim - _TSIM < 0, lru_b, lru_b - 1e9)
  sel = jnp.where(valid, sel, _NEG)

  m = jnp.max(sel, axis=1, keepdims=True)
  jloc = jnp.min(jnp.where(sel == m, coli, 2 ** 30), axis=1, keepdims=True)
  at = coli == jloc
  sim_at = jnp.max(jnp.where(at, sim, _NEG), axis=1, keepdims=True)
  lru_at = jnp.max(jnp.where(at, lru_b, _NEG), axis=1, keepdims=True)
  sc_at = jnp.max(jnp.where(at, jnp.broadcast_to(scv, (_B, _TILE)), _NEG),
                  axis=1, keepdims=True)

  upd = m > bval[...]
  bval[...] = jnp.where(upd, m, bval[...])
  bidx[...] = jnp.where(upd, (pid * _TILE + jloc).astype(jnp.float32),
                        bidx[...])
  bsim[...] = jnp.where(upd, sim_at, bsim[...])
  blru[...] = jnp.where(upd, lru_at, blru[...])
  bsc[...] = jnp.where(upd, sc_at, bsc[...])

  coli1 = lax.broadcasted_iota(jnp.int32, (1, _TILE), 1)
  nnz[...] += jnp.sum(
      jnp.where((pid * _TILE + coli1 < _C) & (usage > 0), 1.0, 0.0),
      axis=(0, 1), keepdims=True)

  @pl.when(pid == _NT - 1)
  def _fin():
    idxf = bidx[...]
    idx_ref[...] = idxf.astype(jnp.int32)
    sim_ref[...] = bsim[...]
    usage_at = 0.1 / blru[...] - 1.0

    ii = lax.broadcasted_iota(jnp.int32, (_B, _B), 0)
    jj = lax.broadcasted_iota(jnp.int32, (_B, _B), 1)
    eyef = jnp.where(ii == jj, 1.0, 0.0)
    idx_row = lax.dot_general(idxf, eyef, (((0,), (0,)), ((), ())),
                              preferred_element_type=jnp.float32,
                              precision=lax.Precision.HIGHEST)
    eqm = idxf == idx_row
    winf = jnp.max(jnp.where(eqm, jj, -1), axis=1, keepdims=True)
    win_ref[...] = winf
    cnt = jnp.sum(jnp.where(eqm, 1.0, 0.0), axis=1, keepdims=True)
    nuse_ref[...] = usage_at + cnt

    own = lax.broadcasted_iota(jnp.int32, (_B, 1), 0)
    is_win = winf == own
    wcol = jnp.where(is_win, idxf, -1.0)
    wcol_ref[...] = wcol
    wrow_ref[...] = lax.dot_general(wcol, eyef, (((0,), (0,)), ((), ())),
                                    preferred_element_type=jnp.float32,
                                    precision=lax.Precision.HIGHEST)

    uniq = jnp.sum(jnp.where(is_win, 1.0, 0.0), axis=(0, 1), keepdims=True)
    was_nz = jnp.sum(jnp.where(is_win & (usage_at > 0), 1.0, 0.0),
                     axis=(0, 1), keepdims=True)
    load_ref[...] = (nnz[...] - was_nz + uniq) / _C

    sumsq = jnp.sum(mv * mv, axis=1, keepdims=True)
    gate = 1.0 / (1.0 + jnp.exp(-sumsq))
    pay_ref[:, 0:_D] = mv
    pay_ref[:, _D:2 * _D] = jnp.broadcast_to(gate, (_B, _D))
    for d in range(_D):
      outer_ref[:, d * _D:(d + 1) * _D] = (mv * mv[:, d:d + 1]) * bsc[...]


_PHASE_A_KWARGS = dict(
    grid=(_NT,),
    in_specs=[
        pl.BlockSpec((_B, _D), lambda i: (0, 0)),
        pl.BlockSpec((_TILE, _D), lambda i: (i, 0)),
        pl.BlockSpec((1, 1, _TILE), lambda i: (i, 0, 0)),
        pl.BlockSpec((1, 1, _TILE), lambda i: (i, 0, 0)),
    ],
    out_specs=[
        pl.BlockSpec((_B, 1), lambda i: (0, 0)),
        pl.BlockSpec((_B, 1), lambda i: (0, 0)),
        pl.BlockSpec((_B, 1), lambda i: (0, 0)),
        pl.BlockSpec((1, _B), lambda i: (0, 0)),
        pl.BlockSpec((_B, 1), lambda i: (0, 0)),
        pl.BlockSpec((_B, 1), lambda i: (0, 0)),
        pl.BlockSpec((_B, 2 * _D), lambda i: (0, 0)),
        pl.BlockSpec((1, 1), lambda i: (0, 0)),
        pl.BlockSpec((_B, _D * _D), lambda i: (0, 0)),
    ],
    out_shape=[
        jax.ShapeDtypeStruct((_B, 1), jnp.int32),
        jax.ShapeDtypeStruct((_B, 1), jnp.float32),
        jax.ShapeDtypeStruct((_B, 1), jnp.int32),
        jax.ShapeDtypeStruct((1, _B), jnp.float32),
        jax.ShapeDtypeStruct((_B, 1), jnp.float32),
        jax.ShapeDtypeStruct((_B, 1), jnp.float32),
        jax.ShapeDtypeStruct((_B, 2 * _D), jnp.float32),
        jax.ShapeDtypeStruct((1, 1), jnp.float32),
        jax.ShapeDtypeStruct((_B, _D * _D), jnp.float32),
    ],
    scratch_shapes=[pltpu.VMEM((_B, 1), jnp.float32)] * 5
    + [pltpu.VMEM((1, 1), jnp.float32)],
    compiler_params=pltpu.CompilerParams(
        dimension_semantics=("arbitrary",)),
)


def _phase_b(pat_ref, gate_ref, usage_ref, wrow_ref, wcol_ref, nusec_ref,
             pay_ref, npat_ref, ngate_ref, nuse_ref):
  pid = pl.program_id(0)
  rowc = pid * _TILE + lax.broadcasted_iota(jnp.int32, (_TILE, 1), 0)
  oh = jnp.where(rowc.astype(jnp.float32) == wrow_ref[...], 1.0, 0.0)
  written = jnp.sum(oh, axis=1, keepdims=True) > 0.0
  gathered = lax.dot_general(oh, pay_ref[...], (((1,), (0,)), ((), ())),
                             preferred_element_type=jnp.float32,
                             precision=lax.Precision.HIGHEST)
  npat_ref[...] = jnp.where(written, gathered[:, 0:_D], pat_ref[...])
  ngate_ref[...] = jnp.where(written, gathered[:, _D:2 * _D], gate_ref[...])

  rowr = pid * _TILE + lax.broadcasted_iota(jnp.int32, (1, _TILE), 1)
  oh2 = jnp.where(wcol_ref[...] == rowr.astype(jnp.float32), 1.0, 0.0)
  nuse_row = lax.dot_general(nusec_ref[...], oh2, (((0,), (0,)), ((), ())),
                             preferred_element_type=jnp.float32,
                             precision=lax.Precision.HIGHEST)
  writ2 = jnp.sum(oh2, axis=0, keepdims=True) > 0.0
  nuse_ref[0] = jnp.where(writ2, nuse_row, usage_ref[0])


_PHASE_B_KWARGS = dict(
    grid=(_NT,),
    in_specs=[
        pl.BlockSpec((_TILE, _D), lambda i: (i, 0)),
        pl.BlockSpec((_TILE, _D), lambda i: (i, 0)),
        pl.BlockSpec((1, 1, _TILE), lambda i: (i, 0, 0)),
        pl.BlockSpec((1, _B), lambda i: (0, 0)),
        pl.BlockSpec((_B, 1), lambda i: (0, 0)),
        pl.BlockSpec((_B, 1), lambda i: (0, 0)),
        pl.BlockSpec((_B, 2 * _D), lambda i: (0, 0)),
    ],
    out_specs=[
        pl.BlockSpec((_TILE, _D), lambda i: (i, 0)),
        pl.BlockSpec((_TILE, _D), lambda i: (i, 0)),
        pl.BlockSpec((1, 1, _TILE), lambda i: (i, 0, 0)),
    ],
    out_shape=[
        jax.ShapeDtypeStruct((_C, _D), jnp.float32),
        jax.ShapeDtypeStruct((_C, _D), jnp.float32),
        jax.ShapeDtypeStruct((_NT, 1, _TILE), jnp.float32),
    ],
    compiler_params=pltpu.CompilerParams(
        dimension_semantics=("arbitrary",)),
)


def _sc_scatter(idx_hbm, win_hbm, outer_hbm, sw_ref, idxv, wv, ob, sem):
  wid = lax.axis_index("s") * 2 + lax.axis_index("c")
  base = wid * _R
  pltpu.sync_copy(idx_hbm.at[pl.ds(base, _R)], idxv)
  pltpu.sync_copy(win_hbm.at[pl.ds(base, _R)], wv)
  pltpu.async_copy(outer_hbm.at[wv], ob, sem).wait()
  pltpu.async_copy(ob, sw_ref.at[idxv], sem).wait()


_scatter_kernel_cache = []


def _get_scatter_kernel():
  if not _scatter_kernel_cache:
    _scatter_kernel_cache.append(pl.kernel(
        _sc_scatter,
        out_type=(),
        mesh=plsc.VectorSubcoreMesh(core_axis_name="c", subcore_axis_name="s"),
        scratch_types=[
            pltpu.VMEM((_R,), jnp.int32),
            pltpu.VMEM((_R,), jnp.int32),
            pltpu.VMEM((_R, _D * _D), jnp.float32),
            pltpu.SemaphoreType.DMA,
        ],
    ))
  return _scatter_kernel_cache[0]


def kernel(memory_vector, memory_patterns, synaptic_weights, synaptic_gates,
           structural_complexity, usage_counts):
  pad = _CPAD - _C
  pat_p = jnp.pad(memory_patterns, ((0, pad), (0, 0)))
  usage_p = jnp.pad(usage_counts, (0, pad)).reshape(_NT, 1, _TILE)
  sc_p = jnp.pad(structural_complexity, (0, pad)).reshape(_NT, 1, _TILE)

  (idx2, sims, win2, wrow, wcol, nusec, payload, load2, outers) = (
      pl.pallas_call(_phase_a, **_PHASE_A_KWARGS)(
          memory_vector, pat_p, usage_p, sc_p))

  idx = idx2.reshape(_B)
  win = win2.reshape(_B)

  sw_ref = jax.new_ref(synaptic_weights.reshape(_C, _D * _D))
  _get_scatter_kernel()(idx, win, outers, sw_ref)

  npat, ngate, nuse_p = pl.pallas_call(_phase_b, **_PHASE_B_KWARGS)(
      memory_patterns, synaptic_gates, usage_p, wrow, wcol, nusec, payload)

  return (idx, sims, npat, sw_ref[...].reshape(_C, _D, _D),
          ngate, nuse_p.reshape(_CPAD)[:_C], load2.reshape(()))


# speedup vs baseline: 1.3969x; 1.3969x over previous
import jax
import jax.numpy as jnp
from jax import lax
from jax.experimental import pallas as pl
from jax.experimental.pallas import tpu as pltpu
from jax.experimental.pallas import tpu_sc as plsc

_B, _D, _C = 1024, 32, 100000
_TSIM = 0.8
_EPS = 1e-8
_TILE = 1024
_NT = 98
_CPAD = _NT * _TILE
_NEG = -3.0e38
_NW = 32
_R = _B // _NW


def _phase_a(mv_ref, pat_ref, usage_ref, sc_ref,
             idx_ref, sim_ref, win_ref, wrow_ref, wcol_ref, nuse_ref,
             pay_ref, load_ref, outer_ref,
             bval, bidx, bsim, blru, bsc, nnz):
  pid = pl.program_id(0)

  @pl.when(pid == 0)
  def _init():
    bval[...] = jnp.full((_B, 1), _NEG, jnp.float32)
    bidx[...] = jnp.zeros((_B, 1), jnp.float32)
    bsim[...] = jnp.zeros((_B, 1), jnp.float32)
    blru[...] = jnp.zeros((_B, 1), jnp.float32)
    bsc[...] = jnp.zeros((_B, 1), jnp.float32)
    nnz[...] = jnp.zeros((1, 1), jnp.float32)

  mv = mv_ref[...]
  vn = mv / jnp.maximum(
      jnp.sqrt(jnp.sum(mv * mv, axis=1, keepdims=True)), _EPS)
  p = pat_ref[...]
  pn = p / jnp.maximum(
      jnp.sqrt(jnp.sum(p * p, axis=1, keepdims=True)), _EPS)
  sim = lax.dot_general(vn, pn, (((1,), (1,)), ((), ())),
                        preferred_element_type=jnp.float32)

  usage = usage_ref[0]
  lru01 = (1.0 / (1.0 + usage)) * 0.1
  scv = sc_ref[0]

  coli1 = lax.broadcasted_iota(jnp.int32, (1, _TILE), 1)
  valid1 = pid * _TILE + coli1 < _C

  selrow = jnp.where(valid1, lru01, _NEG)
  mrow = jnp.max(selrow, axis=1, keepdims=True)
  jrow = jnp.min(jnp.where(selrow == mrow, coli1, 2 ** 30),
                 axis=1, keepdims=True)
  atrow = coli1 == jrow
  lru_r = jnp.max(jnp.where(atrow, lru01, _NEG), axis=1, keepdims=True)
  sc_r = jnp.max(jnp.where(atrow, scv, _NEG), axis=1, keepdims=True)
  simmax = jnp.max(sim)

  @pl.when(simmax < _TSIM)
  def _fast():
    coli = lax.broadcasted_iota(jnp.int32, (_B, _TILE), 1)
    sim_at = jnp.max(jnp.where(coli == jrow, sim, _NEG),
                     axis=1, keepdims=True)
    upd = mrow > bval[...]
    gidxf = (pid * _TILE + jrow).astype(jnp.float32)
    bval[...] = jnp.where(upd, mrow, bval[...])
    bidx[...] = jnp.where(upd, gidxf, bidx[...])
    bsim[...] = jnp.where(upd, sim_at, bsim[...])
    blru[...] = jnp.where(upd, lru_r, blru[...])
    bsc[...] = jnp.where(upd, sc_r, bsc[...])

  @pl.when(simmax >= _TSIM)
  def _slow():
    coli = lax.broadcasted_iota(jnp.int32, (_B, _TILE), 1)
    valid = pid * _TILE + coli < _C
    lru_b = jnp.broadcast_to(lru01, (_B, _TILE))
    sel = jnp.where(sim - _TSIM < 0, lru_b, lru_b - 1e9)
    sel = jnp.where(valid, sel, _NEG)

    m = jnp.max(sel, axis=1, keepdims=True)
    jloc = jnp.min(jnp.where(sel == m, coli, 2 ** 30), axis=1, keepdims=True)
    at = coli == jloc
    sim_at = jnp.max(jnp.where(at, sim, _NEG), axis=1, keepdims=True)
    lru_at = jnp.max(jnp.where(at, lru_b, _NEG), axis=1, keepdims=True)
    sc_at = jnp.max(jnp.where(at, jnp.broadcast_to(scv, (_B, _TILE)), _NEG),
                    axis=1, keepdims=True)

    upd = m > bval[...]
    bval[...] = jnp.where(upd, m, bval[...])
    bidx[...] = jnp.where(upd, (pid * _TILE + jloc).astype(jnp.float32),
                          bidx[...])
    bsim[...] = jnp.where(upd, sim_at, bsim[...])
    blru[...] = jnp.where(upd, lru_at, blru[...])
    bsc[...] = jnp.where(upd, sc_at, bsc[...])

  nnz[...] += jnp.sum(jnp.where(valid1 & (usage > 0), 1.0, 0.0),
                      axis=(0, 1), keepdims=True)

  @pl.when(pid == _NT - 1)
  def _fin():
    idxf = bidx[...]
    idx_ref[...] = idxf.astype(jnp.int32)
    sim_ref[...] = bsim[...]
    usage_at = 0.1 / blru[...] - 1.0

    ii = lax.broadcasted_iota(jnp.int32, (_B, _B), 0)
    jj = lax.broadcasted_iota(jnp.int32, (_B, _B), 1)
    eyef = jnp.where(ii == jj, 1.0, 0.0)
    idx_row = lax.dot_general(idxf, eyef, (((0,), (0,)), ((), ())),
                              preferred_element_type=jnp.float32,
                              precision=lax.Precision.HIGHEST)
    eqm = idxf == idx_row
    winf = jnp.max(jnp.where(eqm, jj, -1), axis=1, keepdims=True)
    win_ref[...] = winf
    cnt = jnp.sum(jnp.where(eqm, 1.0, 0.0), axis=1, keepdims=True)
    nuse_ref[...] = usage_at + cnt

    own = lax.broadcasted_iota(jnp.int32, (_B, 1), 0)
    is_win = winf == own
    wcol = jnp.where(is_win, idxf, -1.0)
    wcol_ref[...] = wcol
    wrow_ref[...] = lax.dot_general(wcol, eyef, (((0,), (0,)), ((), ())),
                                    preferred_element_type=jnp.float32,
                                    precision=lax.Precision.HIGHEST)

    uniq = jnp.sum(jnp.where(is_win, 1.0, 0.0), axis=(0, 1), keepdims=True)
    was_nz = jnp.sum(jnp.where(is_win & (usage_at > 0), 1.0, 0.0),
                     axis=(0, 1), keepdims=True)
    load_ref[...] = (nnz[...] - was_nz + uniq) / _C

    sumsq = jnp.sum(mv * mv, axis=1, keepdims=True)
    gate = 1.0 / (1.0 + jnp.exp(-sumsq))
    pay_ref[:, 0:_D] = mv
    pay_ref[:, _D:2 * _D] = jnp.broadcast_to(gate, (_B, _D))
    for d in range(_D):
      outer_ref[:, d * _D:(d + 1) * _D] = (mv * mv[:, d:d + 1]) * bsc[...]


_PHASE_A_KWARGS = dict(
    grid=(_NT,),
    in_specs=[
        pl.BlockSpec((_B, _D), lambda i: (0, 0)),
        pl.BlockSpec((_TILE, _D), lambda i: (i, 0)),
        pl.BlockSpec((1, 1, _TILE), lambda i: (i, 0, 0)),
        pl.BlockSpec((1, 1, _TILE), lambda i: (i, 0, 0)),
    ],
    out_specs=[
        pl.BlockSpec((_B, 1), lambda i: (0, 0)),
        pl.BlockSpec((_B, 1), lambda i: (0, 0)),
        pl.BlockSpec((_B, 1), lambda i: (0, 0)),
        pl.BlockSpec((1, _B), lambda i: (0, 0)),
        pl.BlockSpec((_B, 1), lambda i: (0, 0)),
        pl.BlockSpec((_B, 1), lambda i: (0, 0)),
        pl.BlockSpec((_B, 2 * _D), lambda i: (0, 0)),
        pl.BlockSpec((1, 1), lambda i: (0, 0)),
        pl.BlockSpec((_B, _D * _D), lambda i: (0, 0)),
    ],
    out_shape=[
        jax.ShapeDtypeStruct((_B, 1), jnp.int32),
        jax.ShapeDtypeStruct((_B, 1), jnp.float32),
        jax.ShapeDtypeStruct((_B, 1), jnp.int32),
        jax.ShapeDtypeStruct((1, _B), jnp.float32),
        jax.ShapeDtypeStruct((_B, 1), jnp.float32),
        jax.ShapeDtypeStruct((_B, 1), jnp.float32),
        jax.ShapeDtypeStruct((_B, 2 * _D), jnp.float32),
        jax.ShapeDtypeStruct((1, 1), jnp.float32),
        jax.ShapeDtypeStruct((_B, _D * _D), jnp.float32),
    ],
    scratch_shapes=[pltpu.VMEM((_B, 1), jnp.float32)] * 5
    + [pltpu.VMEM((1, 1), jnp.float32)],
    compiler_params=pltpu.CompilerParams(
        dimension_semantics=("arbitrary",)),
)


def _phase_b(pat_ref, gate_ref, usage_ref, wrow_ref, wcol_ref, nusec_ref,
             pay_ref, npat_ref, ngate_ref, nuse_ref):
  pid = pl.program_id(0)
  wrow = wrow_ref[...]
  lo = (pid * _TILE).astype(jnp.float32)
  hi = lo + float(_TILE)
  nhit = jnp.sum(jnp.where((wrow >= lo) & (wrow < hi), 1.0, 0.0))

  @pl.when(nhit == 0.0)
  def _copy():
    npat_ref[...] = pat_ref[...]
    ngate_ref[...] = gate_ref[...]
    nuse_ref[0] = usage_ref[0]

  @pl.when(nhit > 0.0)
  def _merge():
    rowc = pid * _TILE + lax.broadcasted_iota(jnp.int32, (_TILE, 1), 0)
    oh = jnp.where(rowc.astype(jnp.float32) == wrow, 1.0, 0.0)
    written = jnp.sum(oh, axis=1, keepdims=True) > 0.0
    gathered = lax.dot_general(oh, pay_ref[...], (((1,), (0,)), ((), ())),
                               preferred_element_type=jnp.float32,
                               precision=lax.Precision.HIGHEST)
    npat_ref[...] = jnp.where(written, gathered[:, 0:_D], pat_ref[...])
    ngate_ref[...] = jnp.where(written, gathered[:, _D:2 * _D], gate_ref[...])

    rowr = pid * _TILE + lax.broadcasted_iota(jnp.int32, (1, _TILE), 1)
    oh2 = jnp.where(wcol_ref[...] == rowr.astype(jnp.float32), 1.0, 0.0)
    nuse_row = lax.dot_general(nusec_ref[...], oh2, (((0,), (0,)), ((), ())),
                               preferred_element_type=jnp.float32,
                               precision=lax.Precision.HIGHEST)
    writ2 = jnp.sum(oh2, axis=0, keepdims=True) > 0.0
    nuse_ref[0] = jnp.where(writ2, nuse_row, usage_ref[0])


_PHASE_B_KWARGS = dict(
    grid=(_NT,),
    in_specs=[
        pl.BlockSpec((_TILE, _D), lambda i: (i, 0)),
        pl.BlockSpec((_TILE, _D), lambda i: (i, 0)),
        pl.BlockSpec((1, 1, _TILE), lambda i: (i, 0, 0)),
        pl.BlockSpec((1, _B), lambda i: (0, 0)),
        pl.BlockSpec((_B, 1), lambda i: (0, 0)),
        pl.BlockSpec((_B, 1), lambda i: (0, 0)),
        pl.BlockSpec((_B, 2 * _D), lambda i: (0, 0)),
    ],
    out_specs=[
        pl.BlockSpec((_TILE, _D), lambda i: (i, 0)),
        pl.BlockSpec((_TILE, _D), lambda i: (i, 0)),
        pl.BlockSpec((1, 1, _TILE), lambda i: (i, 0, 0)),
    ],
    out_shape=[
        jax.ShapeDtypeStruct((_C, _D), jnp.float32),
        jax.ShapeDtypeStruct((_C, _D), jnp.float32),
        jax.ShapeDtypeStruct((_NT, 1, _TILE), jnp.float32),
    ],
    compiler_params=pltpu.CompilerParams(
        dimension_semantics=("arbitrary",)),
)


def _sc_scatter(idx_hbm, win_hbm, outer_hbm, sw_ref, idxv, wv, ob, sem):
  wid = lax.axis_index("s") * 2 + lax.axis_index("c")
  base = wid * _R
  pltpu.sync_copy(idx_hbm.at[pl.ds(base, _R)], idxv)
  pltpu.sync_copy(win_hbm.at[pl.ds(base, _R)], wv)
  pltpu.async_copy(outer_hbm.at[wv], ob, sem).wait()
  pltpu.async_copy(ob, sw_ref.at[idxv], sem).wait()


_scatter_kernel_cache = []


def _get_scatter_kernel():
  if not _scatter_kernel_cache:
    _scatter_kernel_cache.append(pl.kernel(
        _sc_scatter,
        out_type=(),
        mesh=plsc.VectorSubcoreMesh(core_axis_name="c", subcore_axis_name="s"),
        scratch_types=[
            pltpu.VMEM((_R,), jnp.int32),
            pltpu.VMEM((_R,), jnp.int32),
            pltpu.VMEM((_R, _D * _D), jnp.float32),
            pltpu.SemaphoreType.DMA,
        ],
    ))
  return _scatter_kernel_cache[0]


def kernel(memory_vector, memory_patterns, synaptic_weights, synaptic_gates,
           structural_complexity, usage_counts):
  pad = _CPAD - _C
  pat_p = jnp.pad(memory_patterns, ((0, pad), (0, 0)))
  usage_p = jnp.pad(usage_counts, (0, pad)).reshape(_NT, 1, _TILE)
  sc_p = jnp.pad(structural_complexity, (0, pad)).reshape(_NT, 1, _TILE)

  (idx2, sims, win2, wrow, wcol, nusec, payload, load2, outers) = (
      pl.pallas_call(_phase_a, **_PHASE_A_KWARGS)(
          memory_vector, pat_p, usage_p, sc_p))

  idx = idx2.reshape(_B)
  win = win2.reshape(_B)

  sw_ref = jax.new_ref(synaptic_weights.reshape(_C, _D * _D))
  _get_scatter_kernel()(idx, win, outers, sw_ref)

  npat, ngate, nuse_p = pl.pallas_call(_phase_b, **_PHASE_B_KWARGS)(
      memory_patterns, synaptic_gates, usage_p, wrow, wcol, nusec, payload)

  return (idx, sims, npat, sw_ref[...].reshape(_C, _D, _D),
          ngate, nuse_p.reshape(_CPAD)[:_C], load2.reshape(()))


# speedup vs baseline: 1.4819x; 1.0609x over previous
import jax
import jax.numpy as jnp
from jax import lax
from jax.experimental import pallas as pl
from jax.experimental.pallas import tpu as pltpu
from jax.experimental.pallas import tpu_sc as plsc

_B, _D, _C = 1024, 32, 100000
_TSIM = 0.8
_EPS = 1e-8
_TILE = 2048
_NT = 49
_CPAD = _NT * _TILE
_NEG = -3.0e38
_NW = 32
_R = _B // _NW
_CH = 4
_RC = _R // _CH


def _phase_a(mv_ref, pat_ref, usage_ref, sc_ref,
             idx_ref, sim_ref, win_ref, wrow_ref, wcol_ref, nuse_ref,
             pay_ref, load_ref, outer_ref,
             bval, bidx, bsim, blru, bsc, nnz):
  pid = pl.program_id(0)

  @pl.when(pid == 0)
  def _init():
    bval[...] = jnp.full((_B, 1), _NEG, jnp.float32)
    bidx[...] = jnp.zeros((_B, 1), jnp.float32)
    bsim[...] = jnp.zeros((_B, 1), jnp.float32)
    blru[...] = jnp.zeros((_B, 1), jnp.float32)
    bsc[...] = jnp.zeros((_B, 1), jnp.float32)
    nnz[...] = jnp.zeros((1, 1), jnp.float32)

  mv = mv_ref[...]
  vn = mv / jnp.maximum(
      jnp.sqrt(jnp.sum(mv * mv, axis=1, keepdims=True)), _EPS)
  p = pat_ref[...]
  rowi = lax.broadcasted_iota(jnp.int32, (_TILE, 1), 0)
  vrow = pid * _TILE + rowi < _C
  pn = jnp.where(
      vrow,
      p / jnp.maximum(jnp.sqrt(jnp.sum(p * p, axis=1, keepdims=True)), _EPS),
      0.0)
  sim = lax.dot_general(vn, pn, (((1,), (1,)), ((), ())),
                        preferred_element_type=jnp.float32)

  usage = usage_ref[0]
  lru01 = (1.0 / (1.0 + usage)) * 0.1
  scv = sc_ref[0]

  coli1 = lax.broadcasted_iota(jnp.int32, (1, _TILE), 1)
  valid1 = pid * _TILE + coli1 < _C

  selrow = jnp.where(valid1, lru01, _NEG)
  mrow = jnp.max(selrow, axis=1, keepdims=True)
  jrow = jnp.min(jnp.where(selrow == mrow, coli1, 2 ** 30),
                 axis=1, keepdims=True)
  atrow = coli1 == jrow
  lru_r = jnp.max(jnp.where(atrow, lru01, _NEG), axis=1, keepdims=True)
  sc_r = jnp.max(jnp.where(atrow, scv, _NEG), axis=1, keepdims=True)
  simmax = jnp.max(sim)

  anyupd = jnp.max(selrow) > jnp.min(bval[...])

  @pl.when((simmax < _TSIM) & anyupd)
  def _fast():
    coli = lax.broadcasted_iota(jnp.int32, (_B, _TILE), 1)
    sim_at = jnp.max(jnp.where(coli == jrow, sim, _NEG),
                     axis=1, keepdims=True)
    upd = mrow > bval[...]
    gidxf = (pid * _TILE + jrow).astype(jnp.float32)
    bval[...] = jnp.where(upd, mrow, bval[...])
    bidx[...] = jnp.where(upd, gidxf, bidx[...])
    bsim[...] = jnp.where(upd, sim_at, bsim[...])
    blru[...] = jnp.where(upd, lru_r, blru[...])
    bsc[...] = jnp.where(upd, sc_r, bsc[...])

  @pl.when(simmax >= _TSIM)
  def _slow():
    coli = lax.broadcasted_iota(jnp.int32, (_B, _TILE), 1)
    valid = pid * _TILE + coli < _C
    lru_b = jnp.broadcast_to(lru01, (_B, _TILE))
    sel = jnp.where(sim - _TSIM < 0, lru_b, lru_b - 1e9)
    sel = jnp.where(valid, sel, _NEG)

    m = jnp.max(sel, axis=1, keepdims=True)
    jloc = jnp.min(jnp.where(sel == m, coli, 2 ** 30), axis=1, keepdims=True)
    at = coli == jloc
    sim_at = jnp.max(jnp.where(at, sim, _NEG), axis=1, keepdims=True)
    lru_at = jnp.max(jnp.where(at, lru_b, _NEG), axis=1, keepdims=True)
    sc_at = jnp.max(jnp.where(at, jnp.broadcast_to(scv, (_B, _TILE)), _NEG),
                    axis=1, keepdims=True)

    upd = m > bval[...]
    bval[...] = jnp.where(upd, m, bval[...])
    bidx[...] = jnp.where(upd, (pid * _TILE + jloc).astype(jnp.float32),
                          bidx[...])
    bsim[...] = jnp.where(upd, sim_at, bsim[...])
    blru[...] = jnp.where(upd, lru_at, blru[...])
    bsc[...] = jnp.where(upd, sc_at, bsc[...])

  nnz[...] += jnp.sum(jnp.where(valid1 & (usage > 0), 1.0, 0.0),
                      axis=(0, 1), keepdims=True)

  @pl.when(pid == _NT - 1)
  def _fin():
    idxf = bidx[...]
    idx_ref[...] = idxf.astype(jnp.int32)
    sim_ref[...] = bsim[...]
    usage_at = 0.1 / blru[...] - 1.0

    ii = lax.broadcasted_iota(jnp.int32, (_B, _B), 0)
    jj = lax.broadcasted_iota(jnp.int32, (_B, _B), 1)
    eyef = jnp.where(ii == jj, 1.0, 0.0)
    idx_row = lax.dot_general(idxf, eyef, (((0,), (0,)), ((), ())),
                              preferred_element_type=jnp.float32,
                              precision=lax.Precision.HIGHEST)
    eqm = idxf == idx_row
    winf = jnp.max(jnp.where(eqm, jj, -1), axis=1, keepdims=True)
    win_ref[...] = winf
    cnt = jnp.sum(jnp.where(eqm, 1.0, 0.0), axis=1, keepdims=True)
    nuse_ref[...] = usage_at + cnt

    own = lax.broadcasted_iota(jnp.int32, (_B, 1), 0)
    is_win = winf == own
    wcol = jnp.where(is_win, idxf, -1.0)
    wcol_ref[...] = wcol
    wrow_ref[...] = lax.dot_general(wcol, eyef, (((0,), (0,)), ((), ())),
                                    preferred_element_type=jnp.float32,
                                    precision=lax.Precision.HIGHEST)

    uniq = jnp.sum(jnp.where(is_win, 1.0, 0.0), axis=(0, 1), keepdims=True)
    was_nz = jnp.sum(jnp.where(is_win & (usage_at > 0), 1.0, 0.0),
                     axis=(0, 1), keepdims=True)
    load_ref[...] = (nnz[...] - was_nz + uniq) / _C

    sumsq = jnp.sum(mv * mv, axis=1, keepdims=True)
    gate = 1.0 / (1.0 + jnp.exp(-sumsq))
    pay_ref[:, 0:_D] = mv
    pay_ref[:, _D:2 * _D] = jnp.broadcast_to(gate, (_B, _D))
    for d in range(_D):
      outer_ref[:, d * _D:(d + 1) * _D] = (mv * mv[:, d:d + 1]) * bsc[...]


_PHASE_A_KWARGS = dict(
    grid=(_NT,),
    in_specs=[
        pl.BlockSpec((_B, _D), lambda i: (0, 0)),
        pl.BlockSpec((_TILE, _D), lambda i: (i, 0)),
        pl.BlockSpec((1, 1, _TILE), lambda i: (i, 0, 0)),
        pl.BlockSpec((1, 1, _TILE), lambda i: (i, 0, 0)),
    ],
    out_specs=[
        pl.BlockSpec((_B, 1), lambda i: (0, 0)),
        pl.BlockSpec((_B, 1), lambda i: (0, 0)),
        pl.BlockSpec((_B, 1), lambda i: (0, 0)),
        pl.BlockSpec((1, _B), lambda i: (0, 0)),
        pl.BlockSpec((_B, 1), lambda i: (0, 0)),
        pl.BlockSpec((_B, 1), lambda i: (0, 0)),
        pl.BlockSpec((_B, 2 * _D), lambda i: (0, 0)),
        pl.BlockSpec((1, 1), lambda i: (0, 0)),
        pl.BlockSpec((_B, _D * _D), lambda i: (0, 0)),
    ],
    out_shape=[
        jax.ShapeDtypeStruct((_B, 1), jnp.int32),
        jax.ShapeDtypeStruct((_B, 1), jnp.float32),
        jax.ShapeDtypeStruct((_B, 1), jnp.int32),
        jax.ShapeDtypeStruct((1, _B), jnp.float32),
        jax.ShapeDtypeStruct((_B, 1), jnp.float32),
        jax.ShapeDtypeStruct((_B, 1), jnp.float32),
        jax.ShapeDtypeStruct((_B, 2 * _D), jnp.float32),
        jax.ShapeDtypeStruct((1, 1), jnp.float32),
        jax.ShapeDtypeStruct((_B, _D * _D), jnp.float32),
    ],
    scratch_shapes=[pltpu.VMEM((_B, 1), jnp.float32)] * 5
    + [pltpu.VMEM((1, 1), jnp.float32)],
    compiler_params=pltpu.CompilerParams(
        dimension_semantics=("arbitrary",)),
)


def _phase_b(pat_ref, gate_ref, usage_ref, wrow_ref, wcol_ref, nusec_ref,
             pay_ref, npat_ref, ngate_ref, nuse_ref):
  pid = pl.program_id(0)
  wrow = wrow_ref[...]
  lo = (pid * _TILE).astype(jnp.float32)
  hi = lo + float(_TILE)
  nhit = jnp.sum(jnp.where((wrow >= lo) & (wrow < hi), 1.0, 0.0))

  @pl.when(nhit == 0.0)
  def _copy():
    npat_ref[...] = pat_ref[...]
    ngate_ref[...] = gate_ref[...]
    nuse_ref[0] = usage_ref[0]

  @pl.when(nhit > 0.0)
  def _merge():
    rowc = pid * _TILE + lax.broadcasted_iota(jnp.int32, (_TILE, 1), 0)
    oh = jnp.where(rowc.astype(jnp.float32) == wrow, 1.0, 0.0)
    written = jnp.sum(oh, axis=1, keepdims=True) > 0.0
    gathered = lax.dot_general(oh, pay_ref[...], (((1,), (0,)), ((), ())),
                               preferred_element_type=jnp.float32,
                               precision=lax.Precision.HIGHEST)
    npat_ref[...] = jnp.where(written, gathered[:, 0:_D], pat_ref[...])
    ngate_ref[...] = jnp.where(written, gathered[:, _D:2 * _D], gate_ref[...])

    rowr = pid * _TILE + lax.broadcasted_iota(jnp.int32, (1, _TILE), 1)
    oh2 = jnp.where(wcol_ref[...] == rowr.astype(jnp.float32), 1.0, 0.0)
    nuse_row = lax.dot_general(nusec_ref[...], oh2, (((0,), (0,)), ((), ())),
                               preferred_element_type=jnp.float32,
                               precision=lax.Precision.HIGHEST)
    writ2 = jnp.sum(oh2, axis=0, keepdims=True) > 0.0
    nuse_ref[0] = jnp.where(writ2, nuse_row, usage_ref[0])


_PHASE_B_KWARGS = dict(
    grid=(_NT,),
    in_specs=[
        pl.BlockSpec((_TILE, _D), lambda i: (i, 0)),
        pl.BlockSpec((_TILE, _D), lambda i: (i, 0)),
        pl.BlockSpec((1, 1, _TILE), lambda i: (i, 0, 0)),
        pl.BlockSpec((1, _B), lambda i: (0, 0)),
        pl.BlockSpec((_B, 1), lambda i: (0, 0)),
        pl.BlockSpec((_B, 1), lambda i: (0, 0)),
        pl.BlockSpec((_B, 2 * _D), lambda i: (0, 0)),
    ],
    out_specs=[
        pl.BlockSpec((_TILE, _D), lambda i: (i, 0)),
        pl.BlockSpec((_TILE, _D), lambda i: (i, 0)),
        pl.BlockSpec((1, 1, _TILE), lambda i: (i, 0, 0)),
    ],
    out_shape=[
        jax.ShapeDtypeStruct((_C, _D), jnp.float32),
        jax.ShapeDtypeStruct((_C, _D), jnp.float32),
        jax.ShapeDtypeStruct((_NT, 1, _TILE), jnp.float32),
    ],
    compiler_params=pltpu.CompilerParams(
        dimension_semantics=("arbitrary",)),
)


def _sc_scatter(idx_hbm, win_hbm, outer_hbm, sw_in, sw_out,
                idxv, wv, ob, semg, sems):
  del sw_in
  wid = lax.axis_index("s") * 2 + lax.axis_index("c")
  base = wid * _R
  for k in range(_CH):
    pltpu.sync_copy(idx_hbm.at[pl.ds(base + k * _RC, _RC)], idxv.at[k])
    pltpu.sync_copy(win_hbm.at[pl.ds(base + k * _RC, _RC)], wv.at[k])
  gathers = [
      pltpu.async_copy(outer_hbm.at[wv.at[k]], ob.at[k], semg)
      for k in range(_CH)
  ]
  scatters = []
  for k in range(_CH):
    gathers[k].wait()
    scatters.append(pltpu.async_copy(ob.at[k], sw_out.at[idxv.at[k]], sems))
  for s in scatters:
    s.wait()


_scatter_kernel_cache = []


def _get_scatter_kernel():
  if not _scatter_kernel_cache:
    from jax._src.pallas import mpmd as _mpmd
    mesh = plsc.VectorSubcoreMesh(core_axis_name="c", subcore_axis_name="s")
    _scatter_kernel_cache.append(_mpmd._mpmd_map(
        [(mesh, _sc_scatter)],
        out_types=[jax.ShapeDtypeStruct((_C, _D * _D), jnp.float32)],
        input_output_aliases={3: 0},
        scratch_types=[
            pltpu.VMEM((_CH, _RC), jnp.int32),
            pltpu.VMEM((_CH, _RC), jnp.int32),
            pltpu.VMEM((_CH, _RC, _D * _D), jnp.float32),
            pltpu.SemaphoreType.DMA,
            pltpu.SemaphoreType.DMA,
        ],
    ))
  return _scatter_kernel_cache[0]


def kernel(memory_vector, memory_patterns, synaptic_weights, synaptic_gates,
           structural_complexity, usage_counts):
  pad = _CPAD - _C
  usage_p = jnp.pad(usage_counts, (0, pad)).reshape(_NT, 1, _TILE)
  sc_p = jnp.pad(structural_complexity, (0, pad)).reshape(_NT, 1, _TILE)

  (idx2, sims, win2, wrow, wcol, nusec, payload, load2, outers) = (
      pl.pallas_call(_phase_a, **_PHASE_A_KWARGS)(
          memory_vector, memory_patterns, usage_p, sc_p))

  idx = idx2.reshape(_B)
  win = win2.reshape(_B)

  (new_sw,) = _get_scatter_kernel()(
      idx, win, outers, synaptic_weights.reshape(_C, _D * _D))

  npat, ngate, nuse_p = pl.pallas_call(_phase_b, **_PHASE_B_KWARGS)(
      memory_patterns, synaptic_gates, usage_p, wrow, wcol, nusec, payload)

  return (idx, sims, npat, new_sw.reshape(_C, _D, _D),
          ngate, nuse_p.reshape(_CPAD)[:_C], load2.reshape(()))


# speedup vs baseline: 3.7868x; 2.5553x over previous
import jax
import jax.numpy as jnp
from jax import lax
from jax.experimental import pallas as pl
from jax.experimental.pallas import tpu as pltpu

_B, _D, _C = 1024, 32, 100000
_DD = _D * _D
_TSIM = 0.8
_EPS = 1e-8
_TILE = 2048
_NT = 49
_CPAD = _NT * _TILE
_NEG = -3.0e38
_TW = 1024
_NTW = 98


def _phase_a(mv_ref, pat_ref, usage_ref, sc_ref,
             idx_ref, sim_ref, wcol_ref, nuse_ref, pay_ref, load_ref,
             outer_ref,
             bval, bidx, bsim, blru, bsc, nnz):
  pid = pl.program_id(0)

  @pl.when(pid == 0)
  def _init():
    bval[...] = jnp.full((_B, 1), _NEG, jnp.float32)
    bidx[...] = jnp.zeros((_B, 1), jnp.float32)
    bsim[...] = jnp.zeros((_B, 1), jnp.float32)
    blru[...] = jnp.zeros((_B, 1), jnp.float32)
    bsc[...] = jnp.zeros((_B, 1), jnp.float32)
    nnz[...] = jnp.zeros((1, 1), jnp.float32)

  mv = mv_ref[...]
  vn = mv / jnp.maximum(
      jnp.sqrt(jnp.sum(mv * mv, axis=1, keepdims=True)), _EPS)
  p = pat_ref[...]

  coli1 = lax.broadcasted_iota(jnp.int32, (1, _TILE), 1)
  valid1 = pid * _TILE + coli1 < _C

  pn = jnp.where(
      valid1,
      p / jnp.maximum(jnp.sqrt(jnp.sum(p * p, axis=0, keepdims=True)), _EPS),
      0.0)
  sim = lax.dot_general(vn, pn, (((1,), (0,)), ((), ())),
                        preferred_element_type=jnp.float32)

  usage = usage_ref[0]
  lru01 = (1.0 / (1.0 + usage)) * 0.1
  scv = sc_ref[0]

  selrow = jnp.where(valid1, lru01, _NEG)
  mrow = jnp.max(selrow, axis=1, keepdims=True)
  jrow = jnp.min(jnp.where(selrow == mrow, coli1, 2 ** 30),
                 axis=1, keepdims=True)
  atrow = coli1 == jrow
  lru_r = jnp.max(jnp.where(atrow, lru01, _NEG), axis=1, keepdims=True)
  sc_r = jnp.max(jnp.where(atrow, scv, _NEG), axis=1, keepdims=True)
  simmax = jnp.max(sim)

  anyupd = jnp.max(selrow) > jnp.min(bval[...])

  @pl.when((simmax < _TSIM) & anyupd)
  def _fast():
    coli = lax.broadcasted_iota(jnp.int32, (_B, _TILE), 1)
    sim_at = jnp.max(jnp.where(coli == jrow, sim, _NEG),
                     axis=1, keepdims=True)
    upd = mrow > bval[...]
    gidxf = (pid * _TILE + jrow).astype(jnp.float32)
    bval[...] = jnp.where(upd, mrow, bval[...])
    bidx[...] = jnp.where(upd, gidxf, bidx[...])
    bsim[...] = jnp.where(upd, sim_at, bsim[...])
    blru[...] = jnp.where(upd, lru_r, blru[...])
    bsc[...] = jnp.where(upd, sc_r, bsc[...])

  @pl.when(simmax >= _TSIM)
  def _slow():
    coli = lax.broadcasted_iota(jnp.int32, (_B, _TILE), 1)
    valid = pid * _TILE + coli < _C
    lru_b = jnp.broadcast_to(lru01, (_B, _TILE))
    sel = jnp.where(sim - _TSIM < 0, lru_b, lru_b - 1e9)
    sel = jnp.where(valid, sel, _NEG)

    m = jnp.max(sel, axis=1, keepdims=True)
    jloc = jnp.min(jnp.where(sel == m, coli, 2 ** 30), axis=1, keepdims=True)
    at = coli == jloc
    sim_at = jnp.max(jnp.where(at, sim, _NEG), axis=1, keepdims=True)
    lru_at = jnp.max(jnp.where(at, lru_b, _NEG), axis=1, keepdims=True)
    sc_at = jnp.max(jnp.where(at, jnp.broadcast_to(scv, (_B, _TILE)), _NEG),
                    axis=1, keepdims=True)

    upd = m > bval[...]
    bval[...] = jnp.where(upd, m, bval[...])
    bidx[...] = jnp.where(upd, (pid * _TILE + jloc).astype(jnp.float32),
                          bidx[...])
    bsim[...] = jnp.where(upd, sim_at, bsim[...])
    blru[...] = jnp.where(upd, lru_at, blru[...])
    bsc[...] = jnp.where(upd, sc_at, bsc[...])

  nnz[...] += jnp.sum(jnp.where(valid1 & (usage > 0), 1.0, 0.0),
                      axis=(0, 1), keepdims=True)

  @pl.when(pid == _NT - 1)
  def _fin():
    idxf = bidx[...]
    idx_ref[...] = idxf.astype(jnp.int32)
    sim_ref[...] = bsim[...]
    usage_at = 0.1 / blru[...] - 1.0

    ii = lax.broadcasted_iota(jnp.int32, (_B, _B), 0)
    jj = lax.broadcasted_iota(jnp.int32, (_B, _B), 1)
    eyef = jnp.where(ii == jj, 1.0, 0.0)
    idx_row = lax.dot_general(idxf, eyef, (((0,), (0,)), ((), ())),
                              preferred_element_type=jnp.float32,
                              precision=lax.Precision.HIGHEST)
    eqm = idxf == idx_row
    winf = jnp.max(jnp.where(eqm, jj, -1), axis=1, keepdims=True)
    cnt = jnp.sum(jnp.where(eqm, 1.0, 0.0), axis=1, keepdims=True)
    nuse_ref[...] = usage_at + cnt

    own = lax.broadcasted_iota(jnp.int32, (_B, 1), 0)
    is_win = winf == own
    wcol = jnp.where(is_win, idxf, -1.0)
    wcol_ref[...] = wcol

    uniq = jnp.sum(jnp.where(is_win, 1.0, 0.0), axis=(0, 1), keepdims=True)
    was_nz = jnp.sum(jnp.where(is_win & (usage_at > 0), 1.0, 0.0),
                     axis=(0, 1), keepdims=True)
    load_ref[...] = (nnz[...] - was_nz + uniq) / _C

    mvT = lax.dot_general(
        jnp.where(lax.broadcasted_iota(jnp.int32, (_D, _D), 0) ==
                  lax.broadcasted_iota(jnp.int32, (_D, _D), 1), 1.0, 0.0),
        mv, (((1,), (1,)), ((), ())),
        preferred_element_type=jnp.float32,
        precision=lax.Precision.HIGHEST)
    sumsq = jnp.sum(mv * mv, axis=1, keepdims=True)
    gate = 1.0 / (1.0 + jnp.exp(-sumsq))
    gateT = lax.dot_general(gate, eyef, (((0,), (0,)), ((), ())),
                            preferred_element_type=jnp.float32,
                            precision=lax.Precision.HIGHEST)
    bscT = lax.dot_general(bsc[...], eyef, (((0,), (0,)), ((), ())),
                           preferred_element_type=jnp.float32,
                           precision=lax.Precision.HIGHEST)
    pay_ref[0:_D, :] = mvT
    pay_ref[_D:2 * _D, :] = jnp.broadcast_to(gateT, (_D, _B))
    for d in range(_D):
      outer_ref[d * _D:(d + 1) * _D, :] = (mvT * mvT[d:d + 1, :]) * bscT


_PHASE_A_KWARGS = dict(
    grid=(_NT,),
    in_specs=[
        pl.BlockSpec((_B, _D), lambda i: (0, 0)),
        pl.BlockSpec((_D, _TILE), lambda i: (0, i)),
        pl.BlockSpec((1, 1, _TILE), lambda i: (i, 0, 0)),
        pl.BlockSpec((1, 1, _TILE), lambda i: (i, 0, 0)),
    ],
    out_specs=[
        pl.BlockSpec((_B, 1), lambda i: (0, 0)),
        pl.BlockSpec((_B, 1), lambda i: (0, 0)),
        pl.BlockSpec((_B, 1), lambda i: (0, 0)),
        pl.BlockSpec((_B, 1), lambda i: (0, 0)),
        pl.BlockSpec((2 * _D, _B), lambda i: (0, 0)),
        pl.BlockSpec((1, 1), lambda i: (0, 0)),
        pl.BlockSpec((_DD, _B), lambda i: (0, 0)),
    ],
    out_shape=[
        jax.ShapeDtypeStruct((_B, 1), jnp.int32),
        jax.ShapeDtypeStruct((_B, 1), jnp.float32),
        jax.ShapeDtypeStruct((_B, 1), jnp.float32),
        jax.ShapeDtypeStruct((_B, 1), jnp.float32),
        jax.ShapeDtypeStruct((2 * _D, _B), jnp.float32),
        jax.ShapeDtypeStruct((1, 1), jnp.float32),
        jax.ShapeDtypeStruct((_DD, _B), jnp.float32),
    ],
    scratch_shapes=[pltpu.VMEM((_B, 1), jnp.float32)] * 5
    + [pltpu.VMEM((1, 1), jnp.float32)],
    compiler_params=pltpu.CompilerParams(
        dimension_semantics=("arbitrary",)),
)


def _phase_b(pat_ref, gate_ref, usage_ref, wcol_ref, nusec_ref, pay_ref,
             npat_ref, ngate_ref, nuse_ref):
  pid = pl.program_id(0)
  wcol = wcol_ref[...]
  lo = (pid * _TILE).astype(jnp.float32)
  hi = lo + float(_TILE)
  nhit = jnp.sum(jnp.where((wcol >= lo) & (wcol < hi), 1.0, 0.0))

  @pl.when(nhit == 0.0)
  def _copy():
    npat_ref[...] = pat_ref[...]
    ngate_ref[...] = gate_ref[...]
    nuse_ref[0] = usage_ref[0]

  @pl.when(nhit > 0.0)
  def _merge():
    rowr = pid * _TILE + lax.broadcasted_iota(jnp.int32, (1, _TILE), 1)
    oh2 = jnp.where(wcol == rowr.astype(jnp.float32), 1.0, 0.0)
    writ2 = jnp.sum(oh2, axis=0, keepdims=True) > 0.0
    gathered = lax.dot_general(pay_ref[...], oh2, (((1,), (0,)), ((), ())),
                               preferred_element_type=jnp.float32,
                               precision=lax.Precision.HIGHEST)
    npat_ref[...] = jnp.where(writ2, gathered[0:_D, :], pat_ref[...])
    ngate_ref[...] = jnp.where(writ2, gathered[_D:2 * _D, :], gate_ref[...])
    nuse_row = lax.dot_general(nusec_ref[...], oh2, (((0,), (0,)), ((), ())),
                               preferred_element_type=jnp.float32,
                               precision=lax.Precision.HIGHEST)
    nuse_ref[0] = jnp.where(writ2, nuse_row, usage_ref[0])


_PHASE_B_KWARGS = dict(
    grid=(_NT,),
    in_specs=[
        pl.BlockSpec((_D, _TILE), lambda i: (0, i)),
        pl.BlockSpec((_D, _TILE), lambda i: (0, i)),
        pl.BlockSpec((1, 1, _TILE), lambda i: (i, 0, 0)),
        pl.BlockSpec((_B, 1), lambda i: (0, 0)),
        pl.BlockSpec((_B, 1), lambda i: (0, 0)),
        pl.BlockSpec((2 * _D, _B), lambda i: (0, 0)),
    ],
    out_specs=[
        pl.BlockSpec((_D, _TILE), lambda i: (0, i)),
        pl.BlockSpec((_D, _TILE), lambda i: (0, i)),
        pl.BlockSpec((1, 1, _TILE), lambda i: (i, 0, 0)),
    ],
    out_shape=[
        jax.ShapeDtypeStruct((_D, _C), jnp.float32),
        jax.ShapeDtypeStruct((_D, _C), jnp.float32),
        jax.ShapeDtypeStruct((_NT, 1, _TILE), jnp.float32),
    ],
    compiler_params=pltpu.CompilerParams(
        dimension_semantics=("arbitrary",)),
)


def _phase_w(sw_ref, wcol_ref, outer_ref, nsw_ref):
  pid = pl.program_id(0)
  wcol = wcol_ref[...]
  lo = (pid * _TW).astype(jnp.float32)
  hi = lo + float(_TW)
  nhit = jnp.sum(jnp.where((wcol >= lo) & (wcol < hi), 1.0, 0.0))

  @pl.when(nhit == 0.0)
  def _copy():
    nsw_ref[...] = sw_ref[...]

  @pl.when(nhit > 0.0)
  def _merge():
    rowr = pid * _TW + lax.broadcasted_iota(jnp.int32, (1, _TW), 1)
    oh2 = jnp.where(wcol == rowr.astype(jnp.float32), 1.0, 0.0)
    writ2 = jnp.sum(oh2, axis=0, keepdims=True) > 0.0
    gathered = lax.dot_general(outer_ref[...], oh2, (((1,), (0,)), ((), ())),
                               preferred_element_type=jnp.float32,
                               precision=lax.Precision.HIGHEST)
    nsw_ref[...] = jnp.where(writ2, gathered, sw_ref[...])


_PHASE_W_KWARGS = dict(
    grid=(_NTW,),
    in_specs=[
        pl.BlockSpec((_DD, _TW), lambda i: (0, i)),
        pl.BlockSpec((_B, 1), lambda i: (0, 0)),
        pl.BlockSpec((_DD, _B), lambda i: (0, 0)),
    ],
    out_specs=[pl.BlockSpec((_DD, _TW), lambda i: (0, i))],
    out_shape=[jax.ShapeDtypeStruct((_DD, _C), jnp.float32)],
    compiler_params=pltpu.CompilerParams(
        dimension_semantics=("arbitrary",)),
)


def kernel(memory_vector, memory_patterns, synaptic_weights, synaptic_gates,
           structural_complexity, usage_counts):
  pad = _CPAD - _C
  usage_p = jnp.pad(usage_counts, (0, pad)).reshape(_NT, 1, _TILE)
  sc_p = jnp.pad(structural_complexity, (0, pad)).reshape(_NT, 1, _TILE)

  patT = memory_patterns.T
  gateT = synaptic_gates.T
  swT = synaptic_weights.reshape(_C, _DD).T

  (idx2, sims, wcol, nusec, payT, load2, outersT) = (
      pl.pallas_call(_phase_a, **_PHASE_A_KWARGS)(
          memory_vector, patT, usage_p, sc_p))

  npatT, ngateT, nuse_p = pl.pallas_call(_phase_b, **_PHASE_B_KWARGS)(
      patT, gateT, usage_p, wcol, nusec, payT)

  (nswT,) = pl.pallas_call(_phase_w, **_PHASE_W_KWARGS)(swT, wcol, outersT)

  return (idx2.reshape(_B), sims, npatT.T,
          nswT.T.reshape(_C, _D, _D), ngateT.T,
          nuse_p.reshape(_CPAD)[:_C], load2.reshape(()))
